# Initial kernel scaffold; baseline (speedup 1.0000x reference)
import jax
import jax.numpy as jnp
from jax.experimental import pallas as pl


def kernel(token_ids, embeddings):
    raise NotImplementedError("write your pallas kernel here")


# speedup vs baseline: 1.0223x; 1.0223x over previous
import functools

import jax
import jax.numpy as jnp
from jax import lax
from jax.experimental import pallas as pl
from jax.experimental.pallas import tpu as pltpu
from jax.experimental.pallas import tpu_sc as plsc


def _make_gather(V, D, N):
    info = plsc.get_sparse_core_info()
    NC, NS = info.num_cores, info.num_subcores
    NW = NC * NS
    assert N % NW == 0
    per_w = N // NW
    C = 128
    assert per_w % C == 0
    n_chunks = per_w // C

    mesh = plsc.VectorSubcoreMesh(core_axis_name="c", subcore_axis_name="s")

    @functools.partial(
        pl.kernel,
        out_type=jax.ShapeDtypeStruct((N, D), jnp.float32),
        mesh=mesh,
        compiler_params=pltpu.CompilerParams(use_tc_tiling_on_sc=False),
        scratch_types=[
            pltpu.VMEM((n_chunks, C), jnp.int32),
            pltpu.VMEM((C, D), jnp.float32),
            pltpu.SemaphoreType.DMA,
        ],
    )
    def gather_kernel(table_hbm, idx_hbm, out_hbm, idx_v, rows_v, sem):
        wid = lax.axis_index("s") * NC + lax.axis_index("c")
        pltpu.sync_copy(idx_hbm.at[wid], idx_v)
        base = wid * per_w

        def step(j, carry):
            pltpu.async_copy(table_hbm.at[idx_v.at[j]], rows_v, sem).wait()
            pltpu.sync_copy(rows_v, out_hbm.at[pl.ds(base + j * C, C)])
            return carry

        lax.fori_loop(0, n_chunks, step, 0)

    def run(table, idx_flat):
        idx3 = idx_flat.reshape(NW, n_chunks, C)
        return gather_kernel(table, idx3)

    return run


def kernel(token_ids, embeddings):
    B, L = token_ids.shape
    V, D = embeddings.shape
    N = B * L
    run = _make_gather(V, D, N)
    out = run(embeddings, token_ids.reshape(N).astype(jnp.int32))
    return out.reshape(B, L, D)


# speedup vs baseline: 1.1098x; 1.0855x over previous
import functools

import jax
import jax.numpy as jnp
from jax import lax
from jax.experimental import pallas as pl
from jax.experimental.pallas import tpu as pltpu
from jax.experimental.pallas import tpu_sc as plsc


def _make_gather(V, D, N):
    info = plsc.get_sparse_core_info()
    NC, NS = info.num_cores, info.num_subcores
    NW = NC * NS
    assert N % NW == 0
    per_w = N // NW
    C = 128
    K = 10
    GROUP = K * C
    assert per_w % (2 * GROUP) == 0
    n_chunks = per_w // C
    n_groups = per_w // GROUP
    n_pairs = n_groups // 2

    mesh = plsc.VectorSubcoreMesh(core_axis_name="c", subcore_axis_name="s")

    @functools.partial(
        pl.kernel,
        out_type=jax.ShapeDtypeStruct((N, D), jnp.float32),
        mesh=mesh,
        compiler_params=pltpu.CompilerParams(use_tc_tiling_on_sc=False),
        scratch_types=[
            pltpu.VMEM((n_chunks, C), jnp.int32),
            pltpu.VMEM((GROUP, D), jnp.float32),
            pltpu.VMEM((GROUP, D), jnp.float32),
            pltpu.SemaphoreType.DMA,
            pltpu.SemaphoreType.DMA,
            pltpu.SemaphoreType.DMA,
            pltpu.SemaphoreType.DMA,
        ],
    )
    def gather_kernel(table_hbm, idx_hbm, out_hbm, idx_v, buf0, buf1,
                      gsem0, gsem1, ssem0, ssem1):
        wid = lax.axis_index("s") * NC + lax.axis_index("c")
        pltpu.sync_copy(idx_hbm.at[wid], idx_v)
        base = wid * per_w

        def fire(g, buf, gsem):
            for k in range(K):
                pltpu.async_copy(
                    table_hbm.at[idx_v.at[g * K + k]],
                    buf.at[pl.ds(k * C, C)], gsem)

        def drain_gathers(buf, gsem):
            pltpu.make_async_copy(out_hbm.at[pl.ds(0, GROUP)], buf, gsem).wait()

        def store(g, buf, ssem):
            pltpu.async_copy(buf, out_hbm.at[pl.ds(base + g * GROUP, GROUP)],
                             ssem)

        def wait_store(buf, ssem):
            pltpu.make_async_copy(buf, out_hbm.at[pl.ds(0, GROUP)], ssem).wait()

        def pair(p, first, last):
            drain_gathers(buf0, gsem0)
            if not first:
                wait_store(buf1, ssem1)
            fire(2 * p + 1, buf1, gsem1)
            store(2 * p, buf0, ssem0)
            drain_gathers(buf1, gsem1)
            wait_store(buf0, ssem0)
            if not last:
                fire(2 * p + 2, buf0, gsem0)
            store(2 * p + 1, buf1, ssem1)

        fire(0, buf0, gsem0)
        pair(0, first=True, last=(n_pairs == 1))

        def body(p, carry):
            pair(p, first=False, last=False)
            return carry

        if n_pairs > 2:
            lax.fori_loop(1, n_pairs - 1, body, 0)
        if n_pairs > 1:
            pair(n_pairs - 1, first=False, last=True)
        wait_store(buf1, ssem1)

    def run(table, idx_flat):
        idx3 = idx_flat.reshape(NW, n_chunks, C)
        return gather_kernel(table, idx3)

    return run


def kernel(token_ids, embeddings):
    B, L = token_ids.shape
    V, D = embeddings.shape
    N = B * L
    run = _make_gather(V, D, N)
    out = run(embeddings, token_ids.reshape(N).astype(jnp.int32))
    return out.reshape(B, L, D)


# speedup vs baseline: 1.1134x; 1.0033x over previous
import functools

import jax
import jax.numpy as jnp
from jax import lax
from jax.experimental import pallas as pl
from jax.experimental.pallas import tpu as pltpu
from jax.experimental.pallas import tpu_sc as plsc


def _make_gather(V, D, N):
    info = plsc.get_sparse_core_info()
    NC, NS = info.num_cores, info.num_subcores
    NW = NC * NS
    assert N % NW == 0
    per_w = N // NW
    C = 640
    NB = 4
    n_g = per_w // C
    assert per_w % C == 0 and n_g % NB == 0 and n_g // NB >= 3
    n_p = n_g // NB

    mesh = plsc.VectorSubcoreMesh(core_axis_name="c", subcore_axis_name="s")

    @functools.partial(
        pl.kernel,
        out_type=jax.ShapeDtypeStruct((N, D), jnp.float32),
        mesh=mesh,
        compiler_params=pltpu.CompilerParams(use_tc_tiling_on_sc=False),
        scratch_types=[
            pltpu.VMEM((n_g, C), jnp.int32),
        ]
        + [pltpu.VMEM((C, D), jnp.float32) for _ in range(NB)]
        + [pltpu.SemaphoreType.DMA for _ in range(2 * NB)],
    )
    def gather_kernel(table_hbm, idx_hbm, out_hbm, idx_v, *bufs_and_sems):
        bufs = bufs_and_sems[:NB]
        gsems = bufs_and_sems[NB:2 * NB]
        ssems = bufs_and_sems[2 * NB:]
        wid = lax.axis_index("s") * NC + lax.axis_index("c")
        pltpu.sync_copy(idx_hbm.at[wid], idx_v)
        base = wid * per_w

        def fire(g, b):
            pltpu.async_copy(table_hbm.at[idx_v.at[g]], bufs[b], gsems[b])

        def wait_gather(b):
            pltpu.make_async_copy(
                out_hbm.at[pl.ds(0, C)], bufs[b], gsems[b]).wait()

        def store(g, b):
            pltpu.async_copy(
                bufs[b], out_hbm.at[pl.ds(base + g * C, C)], ssems[b])

        def wait_store(b):
            pltpu.make_async_copy(
                bufs[b], out_hbm.at[pl.ds(0, C)], ssems[b]).wait()

        def step(g, b, wait_before_fire, do_fire):
            wait_gather(b)
            store(g, b)
            if do_fire:
                tb = (b + NB - 1) % NB
                if wait_before_fire:
                    wait_store(tb)
                fire(g + NB - 1, tb)

        for b in range(NB - 1):
            fire(b, b)

        for b in range(NB):
            step(b, b, b > 0, True)

        def body(p, carry):
            g0 = p * NB
            for b in range(NB):
                step(g0 + b, b, True, True)
            return carry

        lax.fori_loop(1, n_p - 1, body, 0)

        g0 = (n_p - 1) * NB
        step(g0, 0, True, True)
        for b in range(1, NB):
            step(g0 + b, b, True, False)

        for b in range(NB):
            wait_store(b)

    def run(table, idx_flat):
        idx3 = idx_flat.reshape(NW, n_g, C)
        return gather_kernel(table, idx3)

    return run


def kernel(token_ids, embeddings):
    B, L = token_ids.shape
    V, D = embeddings.shape
    N = B * L
    run = _make_gather(V, D, N)
    out = run(embeddings, token_ids.reshape(N).astype(jnp.int32))
    return out.reshape(B, L, D)


# speedup vs baseline: 1.8065x; 1.6224x over previous
import functools

import jax
import jax.numpy as jnp
from jax import lax
from jax.experimental import pallas as pl
from jax.experimental.pallas import tpu as pltpu
from jax.experimental.pallas import tpu_sc as plsc


def _make_gather(V, D, B, L):
    info = plsc.get_sparse_core_info()
    NC, NS = info.num_cores, info.num_subcores
    NW = NC * NS
    assert B % NW == 0
    rows_pw = B // NW
    R = 16
    NB = 4
    n_g = rows_pw // R
    assert rows_pw % R == 0 and n_g % NB == 0 and n_g // NB >= 3
    n_p = n_g // NB

    mesh = plsc.VectorSubcoreMesh(core_axis_name="c", subcore_axis_name="s")

    @functools.partial(
        pl.kernel,
        out_type=jax.ShapeDtypeStruct((B, L, D), jnp.float32),
        mesh=mesh,
        compiler_params=pltpu.CompilerParams(use_tc_tiling_on_sc=False),
        scratch_types=[
            pltpu.VMEM((rows_pw, L), jnp.int32),
        ]
        + [pltpu.VMEM((R, L, D), jnp.float32) for _ in range(NB)]
        + [pltpu.SemaphoreType.DMA for _ in range(2 * NB)],
    )
    def gather_kernel(table_hbm, idx_hbm, out_hbm, idx_v, *bufs_and_sems):
        bufs = bufs_and_sems[:NB]
        gsems = bufs_and_sems[NB:2 * NB]
        ssems = bufs_and_sems[2 * NB:]
        wid = lax.axis_index("s") * NC + lax.axis_index("c")
        row_base = wid * rows_pw
        pltpu.sync_copy(idx_hbm.at[pl.ds(row_base, rows_pw)], idx_v)

        def fire(g, b):
            for k in range(R):
                pltpu.async_copy(
                    table_hbm.at[idx_v.at[g * R + k]],
                    bufs[b].at[k], gsems[b])

        def wait_gather(b):
            pltpu.make_async_copy(
                out_hbm.at[pl.ds(0, R)], bufs[b], gsems[b]).wait()

        def store(g, b):
            pltpu.async_copy(
                bufs[b], out_hbm.at[pl.ds(row_base + g * R, R)], ssems[b])

        def wait_store(b):
            pltpu.make_async_copy(
                bufs[b], out_hbm.at[pl.ds(0, R)], ssems[b]).wait()

        def step(g, b, wait_before_fire, do_fire):
            wait_gather(b)
            store(g, b)
            if do_fire:
                tb = (b + NB - 1) % NB
                if wait_before_fire:
                    wait_store(tb)
                fire(g + NB - 1, tb)

        for b in range(NB - 1):
            fire(b, b)

        for b in range(NB):
            step(b, b, b > 0, True)

        def body(p, carry):
            g0 = p * NB
            for b in range(NB):
                step(g0 + b, b, True, True)
            return carry

        lax.fori_loop(1, n_p - 1, body, 0)

        g0 = (n_p - 1) * NB
        step(g0, 0, True, True)
        for b in range(1, NB):
            step(g0 + b, b, True, False)

        for b in range(NB):
            wait_store(b)

    return gather_kernel


def kernel(token_ids, embeddings):
    B, L = token_ids.shape
    V, D = embeddings.shape
    run = _make_gather(V, D, B, L)
    return run(embeddings, token_ids.astype(jnp.int32))


# speedup vs baseline: 2.5406x; 1.4064x over previous
import functools

import jax
import jax.numpy as jnp
from jax import lax
from jax.experimental import pallas as pl
from jax.experimental.pallas import tpu as pltpu
from jax.experimental.pallas import tpu_sc as plsc


def _make_gather(V, D, B, L):
    info = plsc.get_sparse_core_info()
    NC, NS = info.num_cores, info.num_subcores
    NW = NC * NS
    assert B % NW == 0
    rows_pw = B // NW
    LP, DP = 56, 128
    R = 16
    NB = 4
    n_g = rows_pw // R
    assert rows_pw % R == 0 and n_g % NB == 0 and n_g // NB >= 3
    n_p = n_g // NB

    mesh = plsc.VectorSubcoreMesh(core_axis_name="c", subcore_axis_name="s")

    @functools.partial(
        pl.kernel,
        out_type=jax.ShapeDtypeStruct((B, LP, DP), jnp.float32),
        mesh=mesh,
        compiler_params=pltpu.CompilerParams(use_tc_tiling_on_sc=False),
        scratch_types=[
            pltpu.VMEM((rows_pw, L), jnp.int32),
        ]
        + [pltpu.VMEM((R, L, D), jnp.float32) for _ in range(NB)]
        + [pltpu.SemaphoreType.DMA for _ in range(2 * NB)],
    )
    def gather_kernel(table_hbm, idx_hbm, out_hbm, idx_v, *bufs_and_sems):
        bufs = bufs_and_sems[:NB]
        gsems = bufs_and_sems[NB:2 * NB]
        ssems = bufs_and_sems[2 * NB:]
        wid = lax.axis_index("s") * NC + lax.axis_index("c")
        row_base = wid * rows_pw
        pltpu.sync_copy(idx_hbm.at[pl.ds(row_base, rows_pw)], idx_v)

        def fire(g, b):
            for k in range(R):
                pltpu.async_copy(
                    table_hbm.at[idx_v.at[g * R + k]],
                    bufs[b].at[k], gsems[b])

        def out_view(r0):
            return out_hbm.at[pl.ds(r0, R), pl.ds(0, L), pl.ds(0, D)]

        def wait_gather(b):
            pltpu.make_async_copy(out_view(0), bufs[b], gsems[b]).wait()

        def store(g, b):
            pltpu.async_copy(bufs[b], out_view(row_base + g * R), ssems[b])

        def wait_store(b):
            pltpu.make_async_copy(bufs[b], out_view(0), ssems[b]).wait()

        def step(g, b, wait_before_fire, do_fire):
            wait_gather(b)
            store(g, b)
            if do_fire:
                tb = (b + NB - 1) % NB
                if wait_before_fire:
                    wait_store(tb)
                fire(g + NB - 1, tb)

        for b in range(NB - 1):
            fire(b, b)

        for b in range(NB):
            step(b, b, b > 0, True)

        def body(p, carry):
            g0 = p * NB
            for b in range(NB):
                step(g0 + b, b, True, True)
            return carry

        lax.fori_loop(1, n_p - 1, body, 0)

        g0 = (n_p - 1) * NB
        step(g0, 0, True, True)
        for b in range(1, NB):
            step(g0 + b, b, True, False)

        for b in range(NB):
            wait_store(b)

    return gather_kernel


def kernel(token_ids, embeddings):
    B, L = token_ids.shape
    V, D = embeddings.shape
    run = _make_gather(V, D, B, L)
    out_pad = run(embeddings, token_ids.astype(jnp.int32))
    return out_pad[:, :L, :D]


# speedup vs baseline: 2.5424x; 1.0007x over previous
import functools

import jax
import jax.numpy as jnp
from jax import lax
from jax.experimental import pallas as pl
from jax.experimental.pallas import tpu as pltpu
from jax.experimental.pallas import tpu_sc as plsc


def _make_gather(V, D, B, L):
    info = plsc.get_sparse_core_info()
    NC, NS = info.num_cores, info.num_subcores
    NW = NC * NS
    assert B % NW == 0
    rows_pw = B // NW
    LP = -(-L // 8) * 8
    DP = -(-D // 128) * 128
    R = 16
    NB = 4
    n_g = rows_pw // R
    assert rows_pw % R == 0 and n_g % NB == 0 and n_g // NB >= 3
    n_p = n_g // NB

    mesh = plsc.VectorSubcoreMesh(core_axis_name="c", subcore_axis_name="s")

    @functools.partial(
        pl.kernel,
        out_type=jax.ShapeDtypeStruct((B, LP, DP), jnp.float32),
        mesh=mesh,
        compiler_params=pltpu.CompilerParams(use_tc_tiling_on_sc=False),
        scratch_types=[
            pltpu.VMEM((rows_pw, L), jnp.int32),
        ]
        + [pltpu.VMEM((R, L, D), jnp.float32) for _ in range(NB)]
        + [pltpu.SemaphoreType.DMA for _ in range(2 * NB)],
    )
    def gather_kernel(table_hbm, idx_hbm, out_hbm, idx_v, *bufs_and_sems):
        bufs = bufs_and_sems[:NB]
        gsems = bufs_and_sems[NB:2 * NB]
        ssems = bufs_and_sems[2 * NB:]
        wid = lax.axis_index("s") * NC + lax.axis_index("c")
        row_base = wid * rows_pw
        pltpu.sync_copy(idx_hbm.at[pl.ds(row_base, rows_pw)], idx_v)

        def fire(g, b):
            for k in range(R):
                pltpu.async_copy(
                    table_hbm.at[idx_v.at[g * R + k]],
                    bufs[b].at[k], gsems[b])

        def out_view(r0):
            return out_hbm.at[pl.ds(r0, R), pl.ds(0, L), pl.ds(0, D)]

        def wait_gather(b):
            pltpu.make_async_copy(out_view(0), bufs[b], gsems[b]).wait()

        def store(g, b):
            pltpu.async_copy(bufs[b], out_view(row_base + g * R), ssems[b])

        def wait_store(b):
            pltpu.make_async_copy(bufs[b], out_view(0), ssems[b]).wait()

        def step(g, b, wait_before_fire, do_fire):
            wait_gather(b)
            store(g, b)
            if do_fire:
                tb = (b + NB - 1) % NB
                if wait_before_fire:
                    wait_store(tb)
                fire(g + NB - 1, tb)

        for b in range(NB - 1):
            fire(b, b)

        for b in range(NB):
            step(b, b, b > 0, True)

        def body(p, carry):
            g0 = p * NB
            for b in range(NB):
                step(g0 + b, b, True, True)
            return carry

        lax.fori_loop(1, n_p - 1, body, 0)

        g0 = (n_p - 1) * NB
        step(g0, 0, True, True)
        for b in range(1, NB):
            step(g0 + b, b, True, False)

        for b in range(NB):
            wait_store(b)

    return gather_kernel


def kernel(token_ids, embeddings):
    B, L = token_ids.shape
    V, D = embeddings.shape
    run = _make_gather(V, D, B, L)
    out_pad = run(embeddings, token_ids.astype(jnp.int32))
    return out_pad[:, :L, :D]
